# baseline (speedup 1.0000x reference)
import functools

import jax
import jax.numpy as jnp
from jax import lax
from jax.experimental import pallas as pl
from jax.experimental.pallas import tpu as pltpu
from jax.experimental.pallas import tpu_sc as plsc

_N = 10000
_HN = _N // 2
_D = 128
_E = 320000
_BN_EPS = 1e-3

_NC = 2
_NS = 16
_NW = _NC * _NS
_K = 32
_EPW = -(-_E // (_NW * _K)) * _K
_EPAD = _NW * _EPW
_NCHUNK = _EPW // _K
_RPT = 624
_APT = 312


def _spmm_body(h_hbm, src_hbm, dst_hbm, vals_hbm, zeros_hbm, out_hbm,
               table, acc, src_v, dst_v, vv_v, dstr_v, vvr_v,
               rows_v, isem, gsem, ssem):
    c = lax.axis_index("c")
    s = lax.axis_index("s")
    wid = s * _NC + c
    tail = _N - _NS * _RPT

    pltpu.sync_copy(h_hbm.at[pl.ds(s * _RPT, _RPT)],
                    table.at[pl.ds(s * _RPT, _RPT)])

    @pl.when(s == _NS - 1)
    def _stage_tail():
        pltpu.sync_copy(h_hbm.at[pl.ds(_NS * _RPT, tail)],
                        table.at[pl.ds(_NS * _RPT, tail)])

    base = wid * _EPW

    def _load(cidx, b):
        off = pl.multiple_of(base + cidx * _K, 8)
        pltpu.async_copy(src_hbm.at[pl.ds(off, _K)], src_v.at[b, 0], isem.at[b])
        pltpu.async_copy(dst_hbm.at[pl.ds(off, _K)], dst_v.at[b, 0], isem.at[b])
        pltpu.async_copy(vals_hbm.at[pl.ds(off, _K)], vv_v.at[b], isem.at[b])

    def _wait_load(b):
        pltpu.make_async_copy(src_hbm.at[pl.ds(0, _K)], src_v.at[b, 0],
                              isem.at[b]).wait()
        pltpu.make_async_copy(dst_hbm.at[pl.ds(0, _K)], dst_v.at[b, 0],
                              isem.at[b]).wait()
        pltpu.make_async_copy(vals_hbm.at[pl.ds(0, _K)], vv_v.at[b],
                              isem.at[b]).wait()

    def _gather(b):
        pltpu.async_copy(table.at[src_v.at[b, 0]], rows_v.at[b], gsem.at[b])

    def _wait_gather(b):
        pltpu.make_async_copy(h_hbm.at[pl.ds(0, _K)], rows_v.at[b],
                              gsem.at[b]).wait()

    def _scatter(b):
        pltpu.async_copy(rows_v.at[b], acc.at[dstr_v.at[b, 0]], ssem.at[b],
                         add=True)

    def _wait_scatter(b):
        pltpu.make_async_copy(h_hbm.at[pl.ds(0, _K)], rows_v.at[b],
                              ssem.at[b]).wait()

    def _scale(b):
        for q in range(_K // 16):
            vv = vvr_v[b, pl.ds(q * 16, 16)]
            for jj in range(16):
                j = q * 16 + jj
                vj = jnp.broadcast_to(vv[jj], (16,))
                for cc in range(_D // 16):
                    sl = pl.ds(cc * 16, 16)
                    rows_v[b, j, sl] = rows_v[b, j, sl] * vj

    def _run_pass(p):
        lo = p * _HN

        pltpu.sync_copy(zeros_hbm.at[pl.ds(0, _APT)],
                        acc.at[pl.ds(s * _APT, _APT)])

        @pl.when(s == _NS - 1)
        def _zero_tail():
            pltpu.sync_copy(zeros_hbm.at[pl.ds(0, _HN - _NS * _APT)],
                            acc.at[pl.ds(_NS * _APT, _HN - _NS * _APT)])

        plsc.subcore_barrier()

        _load(0, 0)
        _load(1, 1)
        _wait_load(0)
        _gather(0)

        def pair_body(g, carry):
            for b in (0, 1):
                cidx = 2 * g + b
                o = 1 - b

                @pl.when(cidx < _NCHUNK)
                def _step():
                    _wait_gather(b)
                    for q in range(_K // 16):
                        sl16 = pl.ds(q * 16, 16)
                        dv = dst_v[b, 0, sl16]
                        vv = vv_v[b, sl16]
                        inr = (dv >= lo) & (dv < lo + _HN)
                        vvr_v[b, sl16] = jnp.where(inr, vv, 0.0)
                        dl = jnp.clip(dv - lo, 0, _HN - 1)
                        dstr_v[b, 0, sl16] = dl

                    _scale(b)

                    @pl.when(cidx + 2 < _NCHUNK)
                    def _():
                        _load(cidx + 2, b)

                    @pl.when(cidx >= 1)
                    def _():
                        _wait_scatter(o)

                    @pl.when(cidx + 1 < _NCHUNK)
                    def _():
                        _wait_load(o)
                        _gather(o)

                    _scatter(b)

            return carry

        lax.fori_loop(0, (_NCHUNK + 1) // 2, pair_body, 0)
        _wait_scatter((_NCHUNK - 1) % 2)
        plsc.subcore_barrier()

        pltpu.sync_copy(acc.at[pl.ds(s * _APT, _APT)],
                        out_hbm.at[c].at[pl.ds(lo + s * _APT, _APT)])

        @pl.when(s == _NS - 1)
        def _write_tail():
            pltpu.sync_copy(acc.at[pl.ds(_NS * _APT, _HN - _NS * _APT)],
                            out_hbm.at[c].at[pl.ds(lo + _NS * _APT,
                                                   _HN - _NS * _APT)])

        plsc.subcore_barrier()

    _run_pass(0)
    _run_pass(1)


_spmm = functools.partial(
    pl.kernel,
    out_type=jax.ShapeDtypeStruct((_NC, _N, _D), jnp.float32),
    mesh=plsc.VectorSubcoreMesh(core_axis_name="c", subcore_axis_name="s"),
    scratch_types=[
        pltpu.VMEM_SHARED((_N, _D), jnp.float32),
        pltpu.VMEM_SHARED((_HN, _D), jnp.float32),
        pltpu.VMEM((2, 1, _K), jnp.int32),
        pltpu.VMEM((2, 1, _K), jnp.int32),
        pltpu.VMEM((2, _K), jnp.float32),
        pltpu.VMEM((2, 1, _K), jnp.int32),
        pltpu.VMEM((2, _K), jnp.float32),
        pltpu.VMEM((2, _K, _D), jnp.float32),
        pltpu.SemaphoreType.DMA((2,)),
        pltpu.SemaphoreType.DMA((2,)),
        pltpu.SemaphoreType.DMA((2,)),
    ],
)(_spmm_body)


_R = 1000
_GRID = _N // _R


def _l2n(t):
    return t * lax.rsqrt(jnp.maximum(jnp.sum(t * t, axis=1, keepdims=True),
                                     1e-12))


def _pre_body(x_ref, g_ref, b_ref, m_ref, v_ref, hn_ref, xn_ref):
    xb = x_ref[...]
    sc = g_ref[...] * lax.rsqrt(v_ref[...] + _BN_EPS)
    hn_ref[...] = (xb - m_ref[...]) * sc + b_ref[...]
    xn_ref[...] = _l2n(xb)


_row_spec = pl.BlockSpec((_R, _D), lambda i: (i, 0))
_par_spec = pl.BlockSpec((1, _D), lambda i: (0, 0))
_w_spec = pl.BlockSpec((_D, _D), lambda i: (0, 0))
_co_spec = pl.BlockSpec((1, 1), lambda i: (0, 0))
_p_spec = pl.BlockSpec((_NC, _R, _D), lambda i: (0, i, 0))

_pre = pl.pallas_call(
    _pre_body,
    grid=(_GRID,),
    in_specs=[_row_spec, _par_spec, _par_spec, _par_spec, _par_spec],
    out_specs=[_row_spec, _row_spec],
    out_shape=[jax.ShapeDtypeStruct((_N, _D), jnp.float32)] * 2,
)


def _mid_body(p_ref, hn_ref, w_ref, b_ref, co_ref, g_ref, be_ref, m_ref,
              v_ref, y_ref, hn2_ref):
    agg = p_ref[0] + p_ref[1] + hn_ref[...] * (co_ref[0, 0] + 1.0)
    y = jnp.tanh(jnp.dot(agg, w_ref[...],
                         preferred_element_type=jnp.float32) + b_ref[...])
    y_ref[...] = y
    sc = g_ref[...] * lax.rsqrt(v_ref[...] + _BN_EPS)
    hn2_ref[...] = (y - m_ref[...]) * sc + be_ref[...]


_mid = pl.pallas_call(
    _mid_body,
    grid=(_GRID,),
    in_specs=[_p_spec, _row_spec, _w_spec, _par_spec, _co_spec,
              _par_spec, _par_spec, _par_spec, _par_spec],
    out_specs=[_row_spec, _row_spec],
    out_shape=[jax.ShapeDtypeStruct((_N, _D), jnp.float32)] * 2,
)


def _fin_body(q_ref, hn2_ref, w_ref, b_ref, co_ref, xn_ref, y1_ref, out_ref):
    agg = q_ref[0] + q_ref[1] + hn2_ref[...] * (co_ref[0, 0] + 1.0)
    y2 = jnp.tanh(jnp.dot(agg, w_ref[...],
                          preferred_element_type=jnp.float32) + b_ref[...])
    cat = jnp.concatenate([xn_ref[...], _l2n(y1_ref[...]), _l2n(y2)], axis=1)
    out_ref[...] = _l2n(cat)


_fin = pl.pallas_call(
    _fin_body,
    grid=(_GRID,),
    in_specs=[_p_spec, _row_spec, _w_spec, _par_spec, _co_spec,
              _row_spec, _row_spec],
    out_specs=pl.BlockSpec((_R, 3 * _D), lambda i: (i, 0)),
    out_shape=jax.ShapeDtypeStruct((_N, 3 * _D), jnp.float32),
)


def kernel(x, edge_index_pos, edge_vals_pos, edge_index_neg, edge_vals_neg,
           gamma1, beta1, mean1, var1, coeff1, W1, b1,
           gamma2, beta2, mean2, var2, coeff2, W2, b2):
    r = lambda a: a.reshape(1, _D)
    zeros_b = jnp.zeros((_APT, _D), jnp.float32)

    def pack(ei, ev):
        pad = _EPAD - _E
        return (jnp.pad(ei[1], (0, pad)), jnp.pad(ei[0], (0, pad)),
                jnp.pad(ev, (0, pad)))

    src_p, dst_p, ev_p = pack(edge_index_pos, edge_vals_pos)
    src_n, dst_n, ev_n = pack(edge_index_neg, edge_vals_neg)

    hn1, xn = _pre(x, r(gamma1), r(beta1), r(mean1), r(var1))

    pp = _spmm(hn1, src_p, dst_p, ev_p, zeros_b)
    pn = _spmm(hn1, src_n, dst_n, ev_n, zeros_b)

    bn2 = (r(gamma2), r(beta2), r(mean2), r(var2))
    y1p, hn2p = _mid(pp, hn1, W1, r(b1), coeff1, *bn2)
    y1n, hn2n = _mid(pn, hn1, W1, r(b1), coeff1, *bn2)

    qp = _spmm(hn2p, src_p, dst_p, ev_p, zeros_b)
    qn = _spmm(hn2n, src_n, dst_n, ev_n, zeros_b)

    out_p = _fin(qp, hn2p, W2, r(b2), coeff2, xn, y1p)
    out_n = _fin(qn, hn2n, W2, r(b2), coeff2, xn, y1n)
    return (out_p, out_n)
